# Initial kernel scaffold; baseline (speedup 1.0000x reference)
import jax
import jax.numpy as jnp
from jax.experimental import pallas as pl


def kernel(output, labels):
    raise NotImplementedError("write your pallas kernel here")


# speedup vs baseline: 1.5671x; 1.5671x over previous
import functools

import jax
import jax.numpy as jnp
from jax import lax
from jax.experimental import pallas as pl
from jax.experimental.pallas import tpu as pltpu
from jax.experimental.pallas import tpu_sc as plsc

NUM_HARD = 2
MARGIN = (0.3, 0.7)
ALPHA = 0.25
GAMMA = 2.0

B = 16
A = 98304
C = 5
N = B * A
K = NUM_HARD * B
ROWS = N * C // 640
RB = 512
GRID = ROWS // RB

NEG_INF = float("-inf")


def _stage_a_body(out_ref, lab_ref, sums_ref, negval_ref, lab0_ref):
    i = pl.program_id(0)
    o = out_ref[...]
    l = lab_ref[...]

    lane = lax.broadcasted_iota(jnp.int32, (RB, 640), 1)
    ch0 = (lane % 5) == 0

    posf = jnp.where(ch0 & (l > 0.5), 1.0, 0.0)
    d = jnp.abs(o - l)
    h = jnp.where(d < 1.0, 0.5 * d * d, d - 0.5)
    sl1 = [jnp.sum(h * pltpu.roll(posf, j, 1)) for j in (1, 2, 3, 4)]

    p_i = lax.broadcasted_iota(jnp.int32, (640, 128), 0)
    g_i = lax.broadcasted_iota(jnp.int32, (640, 128), 1)
    S = jnp.where(p_i == 5 * g_i, 1.0, 0.0).astype(jnp.float32)
    outc = jnp.dot(jnp.where(ch0, o, 0.0), S, preferred_element_type=jnp.float32,
                   precision=lax.Precision.HIGHEST)
    labc = jnp.dot(jnp.where(ch0, l, 0.0), S, preferred_element_type=jnp.float32,
                   precision=lax.Precision.HIGHEST)

    pos = labc > 0.5
    neg = labc < -0.5
    probs = jax.nn.sigmoid(outc)
    lp = jnp.maximum(jnp.log(probs), -100.0)
    lq = jnp.maximum(jnp.log(1.0 - probs), -100.0)
    alpha = jnp.where(labc == 1, ALPHA, 1.0 - ALPHA)
    pt = jnp.where(labc == 1, probs, 1.0 - probs)
    ce = -(labc * lp + (1.0 - labc) * lq)
    omp = 1.0 - pt
    focal_pos_p = jnp.sum(jnp.where(pos, alpha * omp * omp * ce, 0.0))

    n_pos_p = jnp.sum(jnp.where(pos, 1.0, 0.0))
    n_neg_p = jnp.sum(jnp.where(neg, 1.0, 0.0))
    pos_true_p = jnp.sum(jnp.where(pos & (probs >= 0.5), 1.0, 0.0))
    pos_margin_p = jnp.sum(jnp.where(pos & (probs > MARGIN[0]) & (probs < 0.5), 1.0, 0.0))

    negval_ref[...] = jnp.where(neg, outc, NEG_INF)
    lab0_ref[...] = labc

    si = lax.broadcasted_iota(jnp.int32, (8, 128), 0)
    li = lax.broadcasted_iota(jnp.int32, (8, 128), 1)
    vals = (n_pos_p, n_neg_p, sl1[0], sl1[1], sl1[2], sl1[3],
            focal_pos_p, pos_true_p, pos_margin_p)
    vec = jnp.zeros((8, 128), jnp.float32)
    for c, v in enumerate(vals):
        vec = vec + jnp.where((si == 0) & (li == c), v, 0.0)

    @pl.when(i == 0)
    def _():
        sums_ref[...] = vec

    @pl.when(i > 0)
    def _():
        sums_ref[...] += vec


def _stage_a(out2, lab2):
    return pl.pallas_call(
        _stage_a_body,
        grid=(GRID,),
        in_specs=[
            pl.BlockSpec((RB, 640), lambda i: (i, 0)),
            pl.BlockSpec((RB, 640), lambda i: (i, 0)),
        ],
        out_specs=[
            pl.BlockSpec((8, 128), lambda i: (0, 0)),
            pl.BlockSpec((RB, 128), lambda i: (i, 0)),
            pl.BlockSpec((RB, 128), lambda i: (i, 0)),
        ],
        out_shape=[
            jax.ShapeDtypeStruct((8, 128), jnp.float32),
            jax.ShapeDtypeStruct((ROWS, 128), jnp.float32),
            jax.ShapeDtypeStruct((ROWS, 128), jnp.float32),
        ],
    )(out2, lab2)


def _merge16(hi_v, hi_l, lo_v, lo_l):
    c = hi_v >= lo_v
    top_v = jnp.where(c, hi_v, lo_v)
    top_l = jnp.where(c, hi_l, lo_l)
    bot_v = jnp.where(c, lo_v, hi_v)
    bot_l = jnp.where(c, lo_l, hi_l)
    return top_v, top_l, bot_v, bot_l


def _stage_b_make(nw):
    per_w = N // nw
    n_vecs = per_w // 16
    mesh = plsc.VectorSubcoreMesh(core_axis_name="c", subcore_axis_name="s")

    @functools.partial(
        pl.kernel,
        mesh=mesh,
        compiler_params=pltpu.CompilerParams(needs_layout_passes=False),
        out_type=jax.ShapeDtypeStruct((nw, 64), jnp.float32),
        scratch_types=[
            pltpu.VMEM((per_w,), jnp.float32),
            pltpu.VMEM((per_w,), jnp.float32),
            pltpu.VMEM((64,), jnp.float32),
        ],
    )
    def bkern(negval_hbm, lab_hbm, out_hbm, vbuf, lbuf, obuf):
        nc = plsc.get_sparse_core_info().num_cores
        wid = lax.axis_index("s") * nc + lax.axis_index("c")
        base = wid * per_w
        pltpu.sync_copy(negval_hbm.at[pl.ds(base, per_w)], vbuf)
        pltpu.sync_copy(lab_hbm.at[pl.ds(base, per_w)], lbuf)

        ninf16 = jnp.full((16,), NEG_INF, jnp.float32)
        zero16 = jnp.zeros((16,), jnp.float32)

        def body(i, carry):
            v_hi, l_hi, v_lo, l_lo, thr = carry
            x = vbuf[pl.ds(i * 16, 16)]
            xl = lbuf[pl.ds(i * 16, 16)]
            xs, xls = plsc.sort_key_val(x, xl, descending=True)

            def do_merge(c):
                v_hi, l_hi, v_lo, l_lo, _ = c
                rx = lax.rev(xs, (0,))
                rxl = lax.rev(xls, (0,))
                h1, h1l, _, _ = _merge16(v_lo, l_lo, rx, rxl)
                h1s, h1ls = plsc.sort_key_val(h1, h1l, descending=True)
                rh = lax.rev(h1s, (0,))
                rhl = lax.rev(h1ls, (0,))
                nh, nhl, nl, nll = _merge16(v_hi, l_hi, rh, rhl)
                nhs, nhls = plsc.sort_key_val(nh, nhl, descending=True)
                nls, nlls = plsc.sort_key_val(nl, nll, descending=True)
                return (nhs, nhls, nls, nlls, nls[15])

            def skip(c):
                return c

            return lax.cond(xs[0] > thr, do_merge, skip,
                            (v_hi, l_hi, v_lo, l_lo, thr))

        v_hi, l_hi, v_lo, l_lo, _ = lax.fori_loop(
            0, n_vecs, body,
            (ninf16, zero16, ninf16, zero16, jnp.float32(NEG_INF)))

        obuf[pl.ds(0, 16)] = v_hi
        obuf[pl.ds(16, 16)] = v_lo
        obuf[pl.ds(32, 16)] = l_hi
        obuf[pl.ds(48, 16)] = l_lo
        pltpu.sync_copy(obuf, out_hbm.at[wid])

    return bkern


def _stage_c_body(sums_ref, vals_ref, labs_ref, res_ref):
    S = sums_ref[...]
    si = lax.broadcasted_iota(jnp.int32, (8, 128), 0)
    li = lax.broadcasted_iota(jnp.int32, (8, 128), 1)

    def pick(c):
        return jnp.sum(jnp.where((si == 0) & (li == c), S, 0.0))

    n_pos = pick(0)
    n_neg = pick(1)
    sl1 = [pick(2), pick(3), pick(4), pick(5)]
    focal_pos_sum = pick(6)
    pos_true = pick(7)
    pos_margin = pick(8)

    vals = vals_ref[...]
    labs = labs_ref[...]
    lin = (lax.broadcasted_iota(jnp.int32, (32, 32), 0) * 32
           + lax.broadcasted_iota(jnp.int32, (32, 32), 1))
    lane = lax.broadcasted_iota(jnp.int32, (8, 128), 1)

    def body(t, carry):
        alive, selv, sell = carry
        masked = jnp.where(alive > 0.0, vals, NEG_INF)
        m = jnp.max(masked)
        cidx = jnp.min(jnp.where(masked == m, lin, jnp.int32(2 ** 30)))
        lab_sel = jnp.sum(jnp.where(lin == cidx, labs, 0.0))
        alive = jnp.where(lin == cidx, 0.0, alive)
        selv = jnp.where(lane == t, m, selv)
        sell = jnp.where(lane == t, lab_sel, sell)
        return alive, selv, sell

    alive0 = jnp.ones((32, 32), jnp.float32)
    selv0 = jnp.full((8, 128), NEG_INF, jnp.float32)
    sell0 = jnp.zeros((8, 128), jnp.float32)
    _, selv, sell = lax.fori_loop(0, K, body, (alive0, selv0, sell0))

    k_count = jnp.minimum(jnp.float32(K), n_neg)
    valid = (lane.astype(jnp.float32) < k_count) & (lane < K) & (si == 0)
    prob = jax.nn.sigmoid(selv)
    labp1 = sell + 1.0
    alpha = jnp.where(labp1 == 1, ALPHA, 1.0 - ALPHA)
    pt = jnp.where(labp1 == 1, prob, 1.0 - prob)
    lp = jnp.maximum(jnp.log(prob), -100.0)
    lq = jnp.maximum(jnp.log(1.0 - prob), -100.0)
    ce = -(labp1 * lp + (1.0 - labp1) * lq)
    omp = 1.0 - pt
    terms = alpha * omp * omp * ce
    focal_neg = jnp.sum(jnp.where(valid, terms, 0.0)) / k_count
    focal_pos = focal_pos_sum / n_pos
    classify = jnp.where(n_pos > 0, 0.5 * focal_pos + 0.5 * focal_neg,
                         0.5 * focal_neg)
    rl = [jnp.where(n_pos > 0, s / n_pos, 0.0) for s in sl1]
    loss = classify + rl[0] + rl[1] + rl[2] + rl[3]
    neg_false = jnp.sum(jnp.where(valid & (prob < 0.5), 1.0, 0.0))
    neg_margin = jnp.sum(jnp.where(valid & (prob > 0.5) & (prob < MARGIN[1]),
                                   1.0, 0.0))

    outs = (loss, classify, rl[0], rl[1], rl[2], rl[3], pos_true, n_pos,
            neg_false, k_count, pos_margin, neg_margin)
    vec = jnp.zeros((8, 128), jnp.float32)
    for c, v in enumerate(outs):
        vec = vec + jnp.where((si == 0) & (li == c), v, 0.0)
    res_ref[...] = vec


def _stage_c(sums, cvals, clabs):
    return pl.pallas_call(
        _stage_c_body,
        out_shape=jax.ShapeDtypeStruct((8, 128), jnp.float32),
    )(sums, cvals, clabs)


def kernel(output, labels):
    out2 = output.reshape(ROWS, 640)
    lab2 = labels.reshape(ROWS, 640)
    sums, negval, lab0 = _stage_a(out2, lab2)

    info = plsc.get_sparse_core_info()
    nw = info.num_cores * info.num_subcores
    cands = _stage_b_make(nw)(negval.reshape(N), lab0.reshape(N))
    cvals = cands[:, 0:32].reshape(32, 32)
    clabs = cands[:, 32:64].reshape(32, 32)

    r = _stage_c(sums, cvals, clabs)
    i32 = jnp.int32
    return (r[0, 0], r[0, 1], r[0, 2], r[0, 3], r[0, 4], r[0, 5],
            r[0, 6].astype(i32), r[0, 7].astype(i32), r[0, 8].astype(i32),
            r[0, 9].astype(i32), jnp.asarray(N, dtype=i32),
            r[0, 10].astype(i32), r[0, 11].astype(i32))
